# baseline (speedup 1.0000x reference)
import functools

import jax
import jax.numpy as jnp
from jax import lax
from jax.experimental import pallas as pl
from jax.experimental.pallas import tpu as pltpu
from jax.experimental.pallas import tpu_sc as plsc

N = 10000
E = 320000
IN_F = 128
HID = 64
OUT_F = 40
OUT_P = 48

NC, NS = 2, 16
NW = NC * NS
ROWS_PER_S = 640
NP = NS * ROWS_PER_S
SENT = N + 100
CHUNK = 128
CH_PER_W = 80
NCHUNK = NW * CH_PER_W
EPAD = NCHUNK * CHUNK
NBUF = 4

_MESH = dict(core_axis_name="c", subcore_axis_name="s")
_SC_PARAMS = pltpu.CompilerParams(use_tc_tiling_on_sc=False)


def _worker_ids():
    cid = lax.axis_index("c")
    sid = lax.axis_index("s")
    return cid, sid, sid * NC + cid


@functools.partial(
    pl.kernel,
    out_type=jax.ShapeDtypeStruct((2, NC, NP), jnp.float32),
    mesh=plsc.VectorSubcoreMesh(**_MESH),
    compiler_params=_SC_PARAMS,
    scratch_types=[
        pltpu.VMEM((CH_PER_W, CHUNK), jnp.int32),
        pltpu.VMEM((CH_PER_W, CHUNK), jnp.int32),
        pltpu.VMEM((CHUNK,), jnp.float32),
        pltpu.VMEM((ROWS_PER_S,), jnp.float32),
        pltpu.SemaphoreType.DMA,
        pltpu.SemaphoreType.DMA,
        pltpu.VMEM_SHARED((NP,), jnp.float32),
        pltpu.VMEM_SHARED((NP,), jnp.float32),
    ],
)
def _deg_kernel(ei_hbm, degs_hbm, isrc_v, idst_v, ones_v, zeros_v,
                sem_s, sem_d, dout_sh, din_sh):
    cid, sid, wid = _worker_ids()
    pltpu.async_copy(ei_hbm.at[0, pl.ds(wid * CH_PER_W, CH_PER_W)], isrc_v,
                     sem_s)
    pltpu.async_copy(ei_hbm.at[1, pl.ds(wid * CH_PER_W, CH_PER_W)], idst_v,
                     sem_d)

    def fill_ones(i, _):
        ones_v[pl.ds(i * 16, 16)] = jnp.ones((16,), jnp.float32)
        return 0

    lax.fori_loop(0, CHUNK // 16, fill_ones, 0)

    def fill_zeros(i, _):
        zeros_v[pl.ds(i * 16, 16)] = jnp.zeros((16,), jnp.float32)
        return 0

    lax.fori_loop(0, ROWS_PER_S // 16, fill_zeros, 0)

    base = sid * ROWS_PER_S
    pltpu.sync_copy(zeros_v, dout_sh.at[pl.ds(base, ROWS_PER_S)])
    pltpu.sync_copy(zeros_v, din_sh.at[pl.ds(base, ROWS_PER_S)])
    pltpu.make_async_copy(ei_hbm.at[0, pl.ds(0, CH_PER_W)], isrc_v,
                          sem_s).wait()
    pltpu.make_async_copy(ei_hbm.at[1, pl.ds(0, CH_PER_W)], idst_v,
                          sem_d).wait()
    plsc.subcore_barrier()

    def chunk_body(k, _):
        pltpu.async_copy(ones_v, dout_sh.at[isrc_v.at[k]], sem_s, add=True)
        pltpu.async_copy(ones_v, din_sh.at[idst_v.at[k]], sem_d, add=True)
        pltpu.make_async_copy(ones_v, dout_sh.at[isrc_v.at[k]], sem_s).wait()
        pltpu.make_async_copy(ones_v, din_sh.at[idst_v.at[k]], sem_d).wait()
        return 0

    lax.fori_loop(0, CH_PER_W, chunk_body, 0)
    plsc.subcore_barrier()
    pltpu.sync_copy(dout_sh.at[pl.ds(base, ROWS_PER_S)],
                    degs_hbm.at[0, cid, pl.ds(base, ROWS_PER_S)])
    pltpu.sync_copy(din_sh.at[pl.ds(base, ROWS_PER_S)],
                    degs_hbm.at[1, cid, pl.ds(base, ROWS_PER_S)])


def _make_agg(F):
    @functools.partial(
        pl.kernel,
        out_type=jax.ShapeDtypeStruct((NC, NP, F), jnp.float32),
        mesh=plsc.VectorSubcoreMesh(**_MESH),
        compiler_params=_SC_PARAMS,
        scratch_types=(
            [pltpu.VMEM((CH_PER_W, CHUNK), jnp.int32),
             pltpu.VMEM((CH_PER_W, CHUNK), jnp.int32)]
            + [pltpu.VMEM((CHUNK, F), jnp.float32) for _ in range(NBUF)]
            + [pltpu.SemaphoreType.DMA for _ in range(NBUF)]
            + [pltpu.SemaphoreType.DMA,
               pltpu.VMEM_SHARED((NP, F), jnp.float32)]
        ),
    )
    def _agg_kernel(h_hbm, ei_hbm, out_hbm, isrc_v, idst_v, *rest):
        msgs = rest[:NBUF]
        gsems = rest[NBUF:2 * NBUF]
        isem = rest[2 * NBUF]
        agg_sh = rest[2 * NBUF + 1]
        cid, sid, wid = _worker_ids()
        pltpu.async_copy(ei_hbm.at[0, pl.ds(wid * CH_PER_W, CH_PER_W)],
                         isrc_v, isem)
        pltpu.async_copy(ei_hbm.at[1, pl.ds(wid * CH_PER_W, CH_PER_W)],
                         idst_v, isem)

        def fill_zeros(r, _):
            for l in range(F // 16):
                msgs[0][r, pl.ds(l * 16, 16)] = jnp.zeros((16,), jnp.float32)
            return 0

        lax.fori_loop(0, CHUNK, fill_zeros, 0)
        base = sid * ROWS_PER_S
        for t in range(ROWS_PER_S // CHUNK):
            pltpu.sync_copy(msgs[0], agg_sh.at[pl.ds(base + t * CHUNK, CHUNK)])
        pltpu.make_async_copy(ei_hbm.at[0, pl.ds(0, CH_PER_W)], isrc_v,
                              isem).wait()
        pltpu.make_async_copy(ei_hbm.at[1, pl.ds(0, CH_PER_W)], idst_v,
                              isem).wait()
        plsc.subcore_barrier()

        for j in range(NBUF):
            pltpu.async_copy(h_hbm.at[isrc_v.at[j]], msgs[j], gsems[j])

        def chunk_body(i, _):
            k0 = i * NBUF
            for j in range(NBUF):
                pltpu.make_async_copy(h_hbm.at[isrc_v.at[k0 + j]], msgs[j],
                                      gsems[j]).wait()
                pltpu.sync_copy(msgs[j], agg_sh.at[idst_v.at[k0 + j]],
                                add=True)

                @pl.when(i < CH_PER_W // NBUF - 1)
                def _():
                    pltpu.async_copy(h_hbm.at[isrc_v.at[k0 + NBUF + j]],
                                     msgs[j], gsems[j])

            return 0

        lax.fori_loop(0, CH_PER_W // NBUF, chunk_body, 0)
        plsc.subcore_barrier()
        pltpu.sync_copy(agg_sh.at[pl.ds(base, ROWS_PER_S)],
                        out_hbm.at[cid, pl.ds(base, ROWS_PER_S)])

    return _agg_kernel


_agg_hid = _make_agg(HID)
_agg_out = _make_agg(OUT_P)


def _mm1_body(x_ref, w_ref, o_ref):
    o_ref[...] = jnp.dot(x_ref[...], w_ref[...],
                         preferred_element_type=jnp.float32)


def _scale1_body(u_ref, degs_ref, o_ref):
    d = degs_ref[0, 0, :N] + degs_ref[0, 1, :N]
    s = lax.rsqrt(jnp.maximum(d, 1.0))
    o_ref[...] = u_ref[...] * s[:, None]


def _mid_body(a_ref, degs_ref, b1_ref, w2_ref, o_ref):
    agg = a_ref[0, :N, :] + a_ref[1, :N, :]
    din = degs_ref[1, 0, :N] + degs_ref[1, 1, :N]
    dout = degs_ref[0, 0, :N] + degs_ref[0, 1, :N]
    si = lax.rsqrt(jnp.maximum(din, 1.0))
    so = lax.rsqrt(jnp.maximum(dout, 1.0))
    t = jnp.maximum(agg * si[:, None] + b1_ref[0, :][None, :], 0.0)
    o_ref[...] = jnp.dot(t, w2_ref[...],
                         preferred_element_type=jnp.float32) * so[:, None]


def _out_body(a_ref, degs_ref, b2_ref, o_ref):
    agg = a_ref[0, :N, :OUT_F] + a_ref[1, :N, :OUT_F]
    din = degs_ref[1, 0, :N] + degs_ref[1, 1, :N]
    si = lax.rsqrt(jnp.maximum(din, 1.0))
    o_ref[...] = agg * si[:, None] + b2_ref[0, :][None, :]


_mm1 = pl.pallas_call(
    _mm1_body, out_shape=jax.ShapeDtypeStruct((N, HID), jnp.float32))
_scale1 = pl.pallas_call(
    _scale1_body, out_shape=jax.ShapeDtypeStruct((N, HID), jnp.float32))
_mid = pl.pallas_call(
    _mid_body, out_shape=jax.ShapeDtypeStruct((N, OUT_P), jnp.float32))
_out = pl.pallas_call(
    _out_body, out_shape=jax.ShapeDtypeStruct((N, OUT_F), jnp.float32))


def kernel(features, edge_index, W1, b1, W2, b2):
    ei = edge_index.astype(jnp.int32)
    npad = EPAD - E
    pad_agg = jnp.concatenate(
        [jnp.zeros((1, npad), jnp.int32),
         jnp.full((1, npad), SENT, jnp.int32)], axis=0)
    ei_agg = jnp.concatenate([ei, pad_agg], axis=1).reshape(2, NCHUNK, CHUNK)
    pad_deg = jnp.full((2, npad), SENT, jnp.int32)
    ei_deg = jnp.concatenate([ei, pad_deg], axis=1).reshape(2, NCHUNK, CHUNK)

    degs = _deg_kernel(ei_deg)
    u = _mm1(features, W1)
    h1s = _scale1(u, degs)
    agg1 = _agg_hid(h1s, ei_agg)
    w2p = jnp.pad(W2, ((0, 0), (0, OUT_P - OUT_F)))
    h2s = _mid(agg1, degs, b1.reshape(1, -1), w2p)
    agg2 = _agg_out(h2s, ei_agg)
    return _out(agg2, degs, b2.reshape(1, -1))


# speedup vs baseline: 1.0094x; 1.0094x over previous
import functools

import jax
import jax.numpy as jnp
from jax import lax
from jax.experimental import pallas as pl
from jax.experimental.pallas import tpu as pltpu
from jax.experimental.pallas import tpu_sc as plsc

N = 10000
E = 320000
IN_F = 128
HID = 64
OUT_F = 40
OUT_P = 48

NC, NS = 2, 16
NW = NC * NS
ROWS_PER_S = 640
NP = NS * ROWS_PER_S
SENT = N + 100
CHUNK = 128
CH_PER_W = 80
NCHUNK = NW * CH_PER_W
EPAD = NCHUNK * CHUNK
NBUF = 4

_MESH = dict(core_axis_name="c", subcore_axis_name="s")
_SC_PARAMS = pltpu.CompilerParams(use_tc_tiling_on_sc=False)


def _worker_ids():
    cid = lax.axis_index("c")
    sid = lax.axis_index("s")
    return cid, sid, sid * NC + cid


@functools.partial(
    pl.kernel,
    out_type=jax.ShapeDtypeStruct((2, NC, NP), jnp.float32),
    mesh=plsc.VectorSubcoreMesh(**_MESH),
    compiler_params=_SC_PARAMS,
    scratch_types=[
        pltpu.VMEM((CH_PER_W, CHUNK), jnp.int32),
        pltpu.VMEM((CH_PER_W, CHUNK), jnp.int32),
        pltpu.VMEM((CHUNK,), jnp.float32),
        pltpu.VMEM((ROWS_PER_S,), jnp.float32),
        pltpu.SemaphoreType.DMA,
        pltpu.SemaphoreType.DMA,
        pltpu.VMEM_SHARED((NP,), jnp.float32),
        pltpu.VMEM_SHARED((NP,), jnp.float32),
    ],
)
def _deg_kernel(ei_hbm, degs_hbm, isrc_v, idst_v, ones_v, zeros_v,
                sem_s, sem_d, dout_sh, din_sh):
    cid, sid, wid = _worker_ids()
    pltpu.async_copy(ei_hbm.at[0, pl.ds(wid * CH_PER_W, CH_PER_W)], isrc_v,
                     sem_s)
    pltpu.async_copy(ei_hbm.at[1, pl.ds(wid * CH_PER_W, CH_PER_W)], idst_v,
                     sem_d)

    def fill_ones(i, _):
        ones_v[pl.ds(i * 16, 16)] = jnp.ones((16,), jnp.float32)
        return 0

    lax.fori_loop(0, CHUNK // 16, fill_ones, 0)

    def fill_zeros(i, _):
        zeros_v[pl.ds(i * 16, 16)] = jnp.zeros((16,), jnp.float32)
        return 0

    lax.fori_loop(0, ROWS_PER_S // 16, fill_zeros, 0)

    base = sid * ROWS_PER_S
    pltpu.sync_copy(zeros_v, dout_sh.at[pl.ds(base, ROWS_PER_S)])
    pltpu.sync_copy(zeros_v, din_sh.at[pl.ds(base, ROWS_PER_S)])
    pltpu.make_async_copy(ei_hbm.at[0, pl.ds(0, CH_PER_W)], isrc_v,
                          sem_s).wait()
    pltpu.make_async_copy(ei_hbm.at[1, pl.ds(0, CH_PER_W)], idst_v,
                          sem_d).wait()
    plsc.subcore_barrier()

    def chunk_body(k, _):
        pltpu.async_copy(ones_v, dout_sh.at[isrc_v.at[k]], sem_s, add=True)
        pltpu.async_copy(ones_v, din_sh.at[idst_v.at[k]], sem_d, add=True)
        pltpu.make_async_copy(ones_v, dout_sh.at[isrc_v.at[k]], sem_s).wait()
        pltpu.make_async_copy(ones_v, din_sh.at[idst_v.at[k]], sem_d).wait()
        return 0

    lax.fori_loop(0, CH_PER_W, chunk_body, 0)
    plsc.subcore_barrier()
    pltpu.sync_copy(dout_sh.at[pl.ds(base, ROWS_PER_S)],
                    degs_hbm.at[0, cid, pl.ds(base, ROWS_PER_S)])
    pltpu.sync_copy(din_sh.at[pl.ds(base, ROWS_PER_S)],
                    degs_hbm.at[1, cid, pl.ds(base, ROWS_PER_S)])


def _make_agg(F):
    @functools.partial(
        pl.kernel,
        out_type=jax.ShapeDtypeStruct((NC, NP, F), jnp.float32),
        mesh=plsc.VectorSubcoreMesh(**_MESH),
        compiler_params=_SC_PARAMS,
        scratch_types=(
            [pltpu.VMEM((CH_PER_W, CHUNK), jnp.int32),
             pltpu.VMEM((CH_PER_W, CHUNK), jnp.int32)]
            + [pltpu.VMEM((CHUNK, F), jnp.float32) for _ in range(NBUF)]
            + [pltpu.SemaphoreType.DMA for _ in range(NBUF)]
            + [pltpu.SemaphoreType.DMA,
               pltpu.VMEM_SHARED((NP, F), jnp.float32)]
        ),
    )
    def _agg_kernel(h_hbm, ei_hbm, out_hbm, isrc_v, idst_v, *rest):
        msgs = rest[:NBUF]
        gsems = rest[NBUF:2 * NBUF]
        isem = rest[2 * NBUF]
        agg_sh = rest[2 * NBUF + 1]
        cid, sid, wid = _worker_ids()
        pltpu.async_copy(ei_hbm.at[0, pl.ds(wid * CH_PER_W, CH_PER_W)],
                         isrc_v, isem)
        pltpu.async_copy(ei_hbm.at[1, pl.ds(wid * CH_PER_W, CH_PER_W)],
                         idst_v, isem)

        def fill_zeros(r, _):
            for l in range(F // 16):
                msgs[0][r, pl.ds(l * 16, 16)] = jnp.zeros((16,), jnp.float32)
            return 0

        lax.fori_loop(0, CHUNK, fill_zeros, 0)
        base = sid * ROWS_PER_S
        for t in range(ROWS_PER_S // CHUNK):
            pltpu.sync_copy(msgs[0], agg_sh.at[pl.ds(base + t * CHUNK, CHUNK)])
        pltpu.make_async_copy(ei_hbm.at[0, pl.ds(0, CH_PER_W)], isrc_v,
                              isem).wait()
        pltpu.make_async_copy(ei_hbm.at[1, pl.ds(0, CH_PER_W)], idst_v,
                              isem).wait()
        plsc.subcore_barrier()

        for j in range(NBUF):
            pltpu.async_copy(h_hbm.at[isrc_v.at[j]], msgs[j], gsems[j])

        def chunk_body(i, _):
            k0 = i * NBUF
            for j in range(NBUF):
                pltpu.make_async_copy(h_hbm.at[isrc_v.at[k0 + j]], msgs[j],
                                      gsems[j]).wait()
                pltpu.sync_copy(msgs[j], agg_sh.at[idst_v.at[k0 + j]],
                                add=True)

                @pl.when(i < CH_PER_W // NBUF - 1)
                def _():
                    pltpu.async_copy(h_hbm.at[isrc_v.at[k0 + NBUF + j]],
                                     msgs[j], gsems[j])

            return 0

        lax.fori_loop(0, CH_PER_W // NBUF, chunk_body, 0)
        plsc.subcore_barrier()
        pltpu.sync_copy(agg_sh.at[pl.ds(base, ROWS_PER_S)],
                        out_hbm.at[cid, pl.ds(base, ROWS_PER_S)])

    return _agg_kernel


_agg_hid = _make_agg(HID)
_agg_out = _make_agg(OUT_P)


def _mm1_body(x_ref, w_ref, o_ref):
    o_ref[...] = jnp.dot(x_ref[...], w_ref[...],
                         preferred_element_type=jnp.float32)


def _scale1_body(u_ref, degs_ref, o_ref):
    d = degs_ref[0, 0, :N] + degs_ref[0, 1, :N]
    s = lax.rsqrt(jnp.maximum(d, 1.0))
    o_ref[...] = u_ref[...] * s[:, None]


def _mid_body(a_ref, degs_ref, b1_ref, w2_ref, o_ref):
    agg = a_ref[0, :N, :] + a_ref[1, :N, :]
    din = degs_ref[1, 0, :N] + degs_ref[1, 1, :N]
    dout = degs_ref[0, 0, :N] + degs_ref[0, 1, :N]
    si = lax.rsqrt(jnp.maximum(din, 1.0))
    so = lax.rsqrt(jnp.maximum(dout, 1.0))
    t = jnp.maximum(agg * si[:, None] + b1_ref[0, :][None, :], 0.0)
    o_ref[...] = jnp.dot(t, w2_ref[...],
                         preferred_element_type=jnp.float32) * so[:, None]


def _out_body(a_ref, degs_ref, b2_ref, o_ref):
    agg = a_ref[0, :N, :OUT_F] + a_ref[1, :N, :OUT_F]
    din = degs_ref[1, 0, :N] + degs_ref[1, 1, :N]
    si = lax.rsqrt(jnp.maximum(din, 1.0))
    o_ref[...] = agg * si[:, None] + b2_ref[0, :][None, :]


_mm1 = pl.pallas_call(
    _mm1_body, out_shape=jax.ShapeDtypeStruct((N, HID), jnp.float32))
_scale1 = pl.pallas_call(
    _scale1_body, out_shape=jax.ShapeDtypeStruct((N, HID), jnp.float32))
_mid = pl.pallas_call(
    _mid_body, out_shape=jax.ShapeDtypeStruct((N, OUT_P), jnp.float32))
_out = pl.pallas_call(
    _out_body, out_shape=jax.ShapeDtypeStruct((N, OUT_F), jnp.float32))


def kernel(features, edge_index, W1, b1, W2, b2):
    ei = edge_index.astype(jnp.int32)
    npad = EPAD - E
    sent = N + (jnp.arange(npad, dtype=jnp.int32) % (NP - N))
    pad_agg = jnp.concatenate(
        [jnp.zeros((1, npad), jnp.int32), sent.reshape(1, npad)], axis=0)
    ei_agg = jnp.concatenate([ei, pad_agg], axis=1).reshape(2, NCHUNK, CHUNK)
    pad_deg = jnp.concatenate(
        [sent.reshape(1, npad), sent.reshape(1, npad)], axis=0)
    ei_deg = jnp.concatenate([ei, pad_deg], axis=1).reshape(2, NCHUNK, CHUNK)

    degs = _deg_kernel(ei_deg)
    u = _mm1(features, W1)
    h1s = _scale1(u, degs)
    agg1 = _agg_hid(h1s, ei_agg)
    w2p = jnp.pad(W2, ((0, 0), (0, OUT_P - OUT_F)))
    h2s = _mid(agg1, degs, b1.reshape(1, -1), w2p)
    agg2 = _agg_out(h2s, ei_agg)
    return _out(agg2, degs, b2.reshape(1, -1))


# speedup vs baseline: 1.0182x; 1.0087x over previous
import functools

import jax
import jax.numpy as jnp
from jax import lax
from jax.experimental import pallas as pl
from jax.experimental.pallas import tpu as pltpu
from jax.experimental.pallas import tpu_sc as plsc

N = 10000
E = 320000
IN_F = 128
HID = 64
OUT_F = 40
OUT_P = 48

NC, NS = 2, 16
NW = NC * NS
ROWS_PER_S = 640
NP = NS * ROWS_PER_S
SENT = N + 100
CHUNK = 128
CH_PER_W = 80
NCHUNK = NW * CH_PER_W
EPAD = NCHUNK * CHUNK
NBUF = 2

_MESH = dict(core_axis_name="c", subcore_axis_name="s")
_SC_PARAMS = pltpu.CompilerParams(use_tc_tiling_on_sc=False)


def _worker_ids():
    cid = lax.axis_index("c")
    sid = lax.axis_index("s")
    return cid, sid, sid * NC + cid


@functools.partial(
    pl.kernel,
    out_type=jax.ShapeDtypeStruct((2, NC, NP), jnp.float32),
    mesh=plsc.VectorSubcoreMesh(**_MESH),
    compiler_params=_SC_PARAMS,
    scratch_types=[
        pltpu.VMEM((CH_PER_W, CHUNK), jnp.int32),
        pltpu.VMEM((CH_PER_W, CHUNK), jnp.int32),
        pltpu.VMEM((CHUNK,), jnp.float32),
        pltpu.VMEM((ROWS_PER_S,), jnp.float32),
        pltpu.SemaphoreType.DMA,
        pltpu.SemaphoreType.DMA,
        pltpu.VMEM_SHARED((NP,), jnp.float32),
        pltpu.VMEM_SHARED((NP,), jnp.float32),
    ],
)
def _deg_kernel(ei_hbm, degs_hbm, isrc_v, idst_v, ones_v, zeros_v,
                sem_s, sem_d, dout_sh, din_sh):
    cid, sid, wid = _worker_ids()
    pltpu.async_copy(ei_hbm.at[0, pl.ds(wid * CH_PER_W, CH_PER_W)], isrc_v,
                     sem_s)
    pltpu.async_copy(ei_hbm.at[1, pl.ds(wid * CH_PER_W, CH_PER_W)], idst_v,
                     sem_d)

    def fill_ones(i, _):
        ones_v[pl.ds(i * 16, 16)] = jnp.ones((16,), jnp.float32)
        return 0

    lax.fori_loop(0, CHUNK // 16, fill_ones, 0)

    def fill_zeros(i, _):
        zeros_v[pl.ds(i * 16, 16)] = jnp.zeros((16,), jnp.float32)
        return 0

    lax.fori_loop(0, ROWS_PER_S // 16, fill_zeros, 0)

    base = sid * ROWS_PER_S
    pltpu.sync_copy(zeros_v, dout_sh.at[pl.ds(base, ROWS_PER_S)])
    pltpu.sync_copy(zeros_v, din_sh.at[pl.ds(base, ROWS_PER_S)])
    pltpu.make_async_copy(ei_hbm.at[0, pl.ds(0, CH_PER_W)], isrc_v,
                          sem_s).wait()
    pltpu.make_async_copy(ei_hbm.at[1, pl.ds(0, CH_PER_W)], idst_v,
                          sem_d).wait()
    plsc.subcore_barrier()

    def chunk_body(k, _):
        pltpu.async_copy(ones_v, dout_sh.at[isrc_v.at[k]], sem_s, add=True)
        pltpu.async_copy(ones_v, din_sh.at[idst_v.at[k]], sem_d, add=True)
        pltpu.make_async_copy(ones_v, dout_sh.at[isrc_v.at[k]], sem_s).wait()
        pltpu.make_async_copy(ones_v, din_sh.at[idst_v.at[k]], sem_d).wait()
        return 0

    lax.fori_loop(0, CH_PER_W, chunk_body, 0)
    plsc.subcore_barrier()
    pltpu.sync_copy(dout_sh.at[pl.ds(base, ROWS_PER_S)],
                    degs_hbm.at[0, cid, pl.ds(base, ROWS_PER_S)])
    pltpu.sync_copy(din_sh.at[pl.ds(base, ROWS_PER_S)],
                    degs_hbm.at[1, cid, pl.ds(base, ROWS_PER_S)])


def _make_agg(F):
    @functools.partial(
        pl.kernel,
        out_type=jax.ShapeDtypeStruct((NC, NP, F), jnp.float32),
        mesh=plsc.VectorSubcoreMesh(**_MESH),
        compiler_params=_SC_PARAMS,
        scratch_types=(
            [pltpu.VMEM((CH_PER_W, CHUNK), jnp.int32),
             pltpu.VMEM((CH_PER_W, CHUNK), jnp.int32)]
            + [pltpu.VMEM((CHUNK, F), jnp.float32) for _ in range(NBUF)]
            + [pltpu.SemaphoreType.DMA for _ in range(NBUF)]
            + [pltpu.SemaphoreType.DMA,
               pltpu.VMEM_SHARED((NP, F), jnp.float32)]
        ),
    )
    def _agg_kernel(h_hbm, ei_hbm, out_hbm, isrc_v, idst_v, *rest):
        msgs = rest[:NBUF]
        gsems = rest[NBUF:2 * NBUF]
        isem = rest[2 * NBUF]
        agg_sh = rest[2 * NBUF + 1]
        cid, sid, wid = _worker_ids()
        pltpu.async_copy(ei_hbm.at[0, pl.ds(wid * CH_PER_W, CH_PER_W)],
                         isrc_v, isem)
        pltpu.async_copy(ei_hbm.at[1, pl.ds(wid * CH_PER_W, CH_PER_W)],
                         idst_v, isem)

        def fill_zeros(r, _):
            for l in range(F // 16):
                msgs[0][r, pl.ds(l * 16, 16)] = jnp.zeros((16,), jnp.float32)
            return 0

        lax.fori_loop(0, CHUNK, fill_zeros, 0)
        base = sid * ROWS_PER_S
        for t in range(ROWS_PER_S // CHUNK):
            pltpu.sync_copy(msgs[0], agg_sh.at[pl.ds(base + t * CHUNK, CHUNK)])
        pltpu.make_async_copy(ei_hbm.at[0, pl.ds(0, CH_PER_W)], isrc_v,
                              isem).wait()
        pltpu.make_async_copy(ei_hbm.at[1, pl.ds(0, CH_PER_W)], idst_v,
                              isem).wait()
        plsc.subcore_barrier()

        for j in range(NBUF):
            pltpu.async_copy(h_hbm.at[isrc_v.at[j]], msgs[j], gsems[j])

        def chunk_body(i, _):
            k0 = i * NBUF
            for j in range(NBUF):
                pltpu.make_async_copy(h_hbm.at[isrc_v.at[k0 + j]], msgs[j],
                                      gsems[j]).wait()
                pltpu.sync_copy(msgs[j], agg_sh.at[idst_v.at[k0 + j]],
                                add=True)

                @pl.when(i < CH_PER_W // NBUF - 1)
                def _():
                    pltpu.async_copy(h_hbm.at[isrc_v.at[k0 + NBUF + j]],
                                     msgs[j], gsems[j])

            return 0

        lax.fori_loop(0, CH_PER_W // NBUF, chunk_body, 0)
        plsc.subcore_barrier()
        pltpu.sync_copy(agg_sh.at[pl.ds(base, ROWS_PER_S)],
                        out_hbm.at[cid, pl.ds(base, ROWS_PER_S)])

    return _agg_kernel


_agg_hid = _make_agg(HID)
_agg_out = _make_agg(OUT_P)


def _mm1_body(x_ref, w_ref, o_ref):
    o_ref[...] = jnp.dot(x_ref[...], w_ref[...],
                         preferred_element_type=jnp.float32)


def _scale1_body(u_ref, degs_ref, o_ref):
    d = degs_ref[0, 0, :N] + degs_ref[0, 1, :N]
    s = lax.rsqrt(jnp.maximum(d, 1.0))
    o_ref[...] = u_ref[...] * s[:, None]


def _mid_body(a_ref, degs_ref, b1_ref, w2_ref, o_ref):
    agg = a_ref[0, :N, :] + a_ref[1, :N, :]
    din = degs_ref[1, 0, :N] + degs_ref[1, 1, :N]
    dout = degs_ref[0, 0, :N] + degs_ref[0, 1, :N]
    si = lax.rsqrt(jnp.maximum(din, 1.0))
    so = lax.rsqrt(jnp.maximum(dout, 1.0))
    t = jnp.maximum(agg * si[:, None] + b1_ref[0, :][None, :], 0.0)
    o_ref[...] = jnp.dot(t, w2_ref[...],
                         preferred_element_type=jnp.float32) * so[:, None]


def _out_body(a_ref, degs_ref, b2_ref, o_ref):
    agg = a_ref[0, :N, :OUT_F] + a_ref[1, :N, :OUT_F]
    din = degs_ref[1, 0, :N] + degs_ref[1, 1, :N]
    si = lax.rsqrt(jnp.maximum(din, 1.0))
    o_ref[...] = agg * si[:, None] + b2_ref[0, :][None, :]


_mm1 = pl.pallas_call(
    _mm1_body, out_shape=jax.ShapeDtypeStruct((N, HID), jnp.float32))
_scale1 = pl.pallas_call(
    _scale1_body, out_shape=jax.ShapeDtypeStruct((N, HID), jnp.float32))
_mid = pl.pallas_call(
    _mid_body, out_shape=jax.ShapeDtypeStruct((N, OUT_P), jnp.float32))
_out = pl.pallas_call(
    _out_body, out_shape=jax.ShapeDtypeStruct((N, OUT_F), jnp.float32))


def kernel(features, edge_index, W1, b1, W2, b2):
    ei = edge_index.astype(jnp.int32)
    npad = EPAD - E
    sent = N + (jnp.arange(npad, dtype=jnp.int32) % (NP - N))
    pad_agg = jnp.concatenate(
        [jnp.zeros((1, npad), jnp.int32), sent.reshape(1, npad)], axis=0)
    ei_agg = jnp.concatenate([ei, pad_agg], axis=1).reshape(2, NCHUNK, CHUNK)
    pad_deg = jnp.concatenate(
        [sent.reshape(1, npad), sent.reshape(1, npad)], axis=0)
    ei_deg = jnp.concatenate([ei, pad_deg], axis=1).reshape(2, NCHUNK, CHUNK)

    degs = _deg_kernel(ei_deg)
    u = _mm1(features, W1)
    h1s = _scale1(u, degs)
    agg1 = _agg_hid(h1s, ei_agg)
    w2p = jnp.pad(W2, ((0, 0), (0, OUT_P - OUT_F)))
    h2s = _mid(agg1, degs, b1.reshape(1, -1), w2p)
    agg2 = _agg_out(h2s, ei_agg)
    return _out(agg2, degs, b2.reshape(1, -1))


# speedup vs baseline: 2.3574x; 2.3153x over previous
import functools

import jax
import jax.numpy as jnp
from jax import lax
from jax.experimental import pallas as pl
from jax.experimental.pallas import tpu as pltpu
from jax.experimental.pallas import tpu_sc as plsc

N = 10000
E = 320000
IN_F = 128
HID = 64
OUT_F = 40
OUT_P = 48

NC, NS = 2, 16
NW = NC * NS
ROWS_PER_S = 640
NP = NS * ROWS_PER_S
SENT = N + 100
CHUNK = 128
CH_PER_W = 80
NCHUNK = NW * CH_PER_W
EPAD = NCHUNK * CHUNK
NBUF = 2

_MESH = dict(core_axis_name="c", subcore_axis_name="s")
_SC_PARAMS = pltpu.CompilerParams(use_tc_tiling_on_sc=False)


def _worker_ids():
    cid = lax.axis_index("c")
    sid = lax.axis_index("s")
    return cid, sid, sid * NC + cid


@functools.partial(
    pl.kernel,
    out_type=jax.ShapeDtypeStruct((2, NC, NP), jnp.float32),
    mesh=plsc.VectorSubcoreMesh(**_MESH),
    compiler_params=_SC_PARAMS,
    scratch_types=[
        pltpu.VMEM((CH_PER_W, CHUNK), jnp.int32),
        pltpu.VMEM((CH_PER_W, CHUNK), jnp.int32),
        pltpu.VMEM((CHUNK,), jnp.float32),
        pltpu.VMEM((ROWS_PER_S,), jnp.float32),
        pltpu.SemaphoreType.DMA,
        pltpu.SemaphoreType.DMA,
        pltpu.VMEM_SHARED((NP,), jnp.float32),
        pltpu.VMEM_SHARED((NP,), jnp.float32),
    ],
)
def _deg_kernel(ei_hbm, degs_hbm, isrc_v, idst_v, ones_v, zeros_v,
                sem_s, sem_d, dout_sh, din_sh):
    cid, sid, wid = _worker_ids()
    pltpu.async_copy(ei_hbm.at[0, pl.ds(wid * CH_PER_W, CH_PER_W)], isrc_v,
                     sem_s)
    pltpu.async_copy(ei_hbm.at[1, pl.ds(wid * CH_PER_W, CH_PER_W)], idst_v,
                     sem_d)

    def fill_ones(i, _):
        ones_v[pl.ds(i * 16, 16)] = jnp.ones((16,), jnp.float32)
        return 0

    lax.fori_loop(0, CHUNK // 16, fill_ones, 0)

    def fill_zeros(i, _):
        zeros_v[pl.ds(i * 16, 16)] = jnp.zeros((16,), jnp.float32)
        return 0

    lax.fori_loop(0, ROWS_PER_S // 16, fill_zeros, 0)

    base = sid * ROWS_PER_S
    pltpu.sync_copy(zeros_v, dout_sh.at[pl.ds(base, ROWS_PER_S)])
    pltpu.sync_copy(zeros_v, din_sh.at[pl.ds(base, ROWS_PER_S)])
    pltpu.make_async_copy(ei_hbm.at[0, pl.ds(0, CH_PER_W)], isrc_v,
                          sem_s).wait()
    pltpu.make_async_copy(ei_hbm.at[1, pl.ds(0, CH_PER_W)], idst_v,
                          sem_d).wait()
    plsc.subcore_barrier()

    def chunk_body(k, _):
        pltpu.async_copy(ones_v, dout_sh.at[isrc_v.at[k]], sem_s, add=True)
        pltpu.async_copy(ones_v, din_sh.at[idst_v.at[k]], sem_d, add=True)
        pltpu.make_async_copy(ones_v, dout_sh.at[isrc_v.at[k]], sem_s).wait()
        pltpu.make_async_copy(ones_v, din_sh.at[idst_v.at[k]], sem_d).wait()
        return 0

    lax.fori_loop(0, CH_PER_W, chunk_body, 0)
    plsc.subcore_barrier()
    pltpu.sync_copy(dout_sh.at[pl.ds(base, ROWS_PER_S)],
                    degs_hbm.at[0, cid, pl.ds(base, ROWS_PER_S)])
    pltpu.sync_copy(din_sh.at[pl.ds(base, ROWS_PER_S)],
                    degs_hbm.at[1, cid, pl.ds(base, ROWS_PER_S)])


def _make_agg(F):
    @functools.partial(
        pl.kernel,
        out_type=jax.ShapeDtypeStruct((NC, NP, F), jnp.float32),
        mesh=plsc.VectorSubcoreMesh(**_MESH),
        compiler_params=_SC_PARAMS,
        scratch_types=(
            [pltpu.VMEM((CH_PER_W, CHUNK), jnp.int32),
             pltpu.VMEM((CH_PER_W, CHUNK), jnp.int32)]
            + [pltpu.VMEM((CHUNK, F), jnp.float32) for _ in range(NBUF)]
            + [pltpu.SemaphoreType.DMA for _ in range(NBUF)]
            + [pltpu.SemaphoreType.DMA,
               pltpu.VMEM_SHARED((NP, F), jnp.float32)]
        ),
    )
    def _agg_kernel(h_hbm, ei_hbm, out_hbm, isrc_v, idst_v, *rest):
        msgs = rest[:NBUF]
        gsems = rest[NBUF:2 * NBUF]
        isem = rest[2 * NBUF]
        agg_sh = rest[2 * NBUF + 1]
        cid, sid, wid = _worker_ids()
        pltpu.async_copy(ei_hbm.at[0, pl.ds(wid * CH_PER_W, CH_PER_W)],
                         isrc_v, isem)
        pltpu.async_copy(ei_hbm.at[1, pl.ds(wid * CH_PER_W, CH_PER_W)],
                         idst_v, isem)

        def fill_zeros(r, _):
            for l in range(F // 16):
                msgs[0][r, pl.ds(l * 16, 16)] = jnp.zeros((16,), jnp.float32)
            return 0

        lax.fori_loop(0, CHUNK, fill_zeros, 0)
        base = sid * ROWS_PER_S
        for t in range(ROWS_PER_S // CHUNK):
            pltpu.sync_copy(msgs[0], agg_sh.at[pl.ds(base + t * CHUNK, CHUNK)])
        pltpu.make_async_copy(ei_hbm.at[0, pl.ds(0, CH_PER_W)], isrc_v,
                              isem).wait()
        pltpu.make_async_copy(ei_hbm.at[1, pl.ds(0, CH_PER_W)], idst_v,
                              isem).wait()
        plsc.subcore_barrier()

        for j in range(NBUF):
            pltpu.async_copy(h_hbm.at[isrc_v.at[j]], msgs[j], gsems[j])

        def chunk_body(i, _):
            k0 = i * NBUF
            for j in range(NBUF):
                pltpu.make_async_copy(h_hbm.at[isrc_v.at[k0 + j]], msgs[j],
                                      gsems[j]).wait()
                pltpu.sync_copy(msgs[j], agg_sh.at[idst_v.at[k0 + j]],
                                add=True)

                @pl.when(i < CH_PER_W // NBUF - 1)
                def _():
                    pltpu.async_copy(h_hbm.at[isrc_v.at[k0 + NBUF + j]],
                                     msgs[j], gsems[j])

            return 0

        lax.fori_loop(0, CH_PER_W // NBUF, chunk_body, 0)
        plsc.subcore_barrier()
        pltpu.sync_copy(agg_sh.at[pl.ds(base, ROWS_PER_S)],
                        out_hbm.at[cid, pl.ds(base, ROWS_PER_S)])

    return _agg_kernel


_agg_hid = _make_agg(HID)
_agg_out = _make_agg(OUT_P)


def _mm1_body(x_ref, w_ref, o_ref):
    o_ref[...] = jnp.dot(x_ref[...], w_ref[...],
                         preferred_element_type=jnp.float32)


def _scale1_body(u_ref, degs_ref, o_ref):
    d = degs_ref[0, 0, :N] + degs_ref[0, 1, :N]
    s = lax.rsqrt(jnp.maximum(d, 1.0))
    o_ref[...] = u_ref[...] * s[:, None]


def _mid_body(a_ref, degs_ref, b1_ref, w2_ref, o_ref):
    agg = a_ref[0, :N, :] + a_ref[1, :N, :]
    din = degs_ref[1, 0, :N] + degs_ref[1, 1, :N]
    dout = degs_ref[0, 0, :N] + degs_ref[0, 1, :N]
    si = lax.rsqrt(jnp.maximum(din, 1.0))
    so = lax.rsqrt(jnp.maximum(dout, 1.0))
    t = jnp.maximum(agg * si[:, None] + b1_ref[0, :][None, :], 0.0)
    o_ref[...] = jnp.dot(t, w2_ref[...],
                         preferred_element_type=jnp.float32) * so[:, None]


def _out_body(a_ref, degs_ref, b2_ref, o_ref):
    agg = a_ref[0, :N, :OUT_F] + a_ref[1, :N, :OUT_F]
    din = degs_ref[1, 0, :N] + degs_ref[1, 1, :N]
    si = lax.rsqrt(jnp.maximum(din, 1.0))
    o_ref[...] = agg * si[:, None] + b2_ref[0, :][None, :]


_mm1 = pl.pallas_call(
    _mm1_body, out_shape=jax.ShapeDtypeStruct((N, HID), jnp.float32))
_scale1 = pl.pallas_call(
    _scale1_body, out_shape=jax.ShapeDtypeStruct((N, HID), jnp.float32))
_mid = pl.pallas_call(
    _mid_body, out_shape=jax.ShapeDtypeStruct((N, OUT_P), jnp.float32))
_out = pl.pallas_call(
    _out_body, out_shape=jax.ShapeDtypeStruct((N, OUT_F), jnp.float32))


def kernel(features, edge_index, W1, b1, W2, b2):
    ei = edge_index.astype(jnp.int32)
    npad = EPAD - E
    sent = N + (jnp.arange(npad, dtype=jnp.int32) % (NP - N))
    pad_src = (jnp.arange(npad, dtype=jnp.int32) * 37) % N
    pad_agg = jnp.concatenate(
        [pad_src.reshape(1, npad), sent.reshape(1, npad)], axis=0)
    ei_agg = jnp.concatenate([ei, pad_agg], axis=1).reshape(2, NCHUNK, CHUNK)
    pad_deg = jnp.concatenate(
        [sent.reshape(1, npad), sent.reshape(1, npad)], axis=0)
    ei_deg = jnp.concatenate([ei, pad_deg], axis=1).reshape(2, NCHUNK, CHUNK)

    degs = _deg_kernel(ei_deg)
    u = _mm1(features, W1)
    h1s = _scale1(u, degs)
    agg1 = _agg_hid(h1s, ei_agg)
    w2p = jnp.pad(W2, ((0, 0), (0, OUT_P - OUT_F)))
    h2s = _mid(agg1, degs, b1.reshape(1, -1), w2p)
    agg2 = _agg_out(h2s, ei_agg)
    return _out(agg2, degs, b2.reshape(1, -1))


# speedup vs baseline: 2.8174x; 1.1951x over previous
import functools

import jax
import jax.numpy as jnp
from jax import lax
from jax.experimental import pallas as pl
from jax.experimental.pallas import tpu as pltpu
from jax.experimental.pallas import tpu_sc as plsc

N = 10000
E = 320000
IN_F = 128
HID = 64
OUT_F = 40
OUT_P = 48

NC, NS = 2, 16
NW = NC * NS
ROWS_PER_S = 640
NP = NS * ROWS_PER_S
SENT = N + 100
CHUNK = 128
CH_PER_W = 80
NCHUNK = NW * CH_PER_W
EPAD = NCHUNK * CHUNK
NBUF = 4

_MESH = dict(core_axis_name="c", subcore_axis_name="s")
_SC_PARAMS = pltpu.CompilerParams(use_tc_tiling_on_sc=False)


def _worker_ids():
    cid = lax.axis_index("c")
    sid = lax.axis_index("s")
    return cid, sid, sid * NC + cid


@functools.partial(
    pl.kernel,
    out_type=jax.ShapeDtypeStruct((2, NC, NP), jnp.float32),
    mesh=plsc.VectorSubcoreMesh(**_MESH),
    compiler_params=_SC_PARAMS,
    scratch_types=[
        pltpu.VMEM((CH_PER_W, CHUNK), jnp.int32),
        pltpu.VMEM((CH_PER_W, CHUNK), jnp.int32),
        pltpu.VMEM((CHUNK,), jnp.float32),
        pltpu.VMEM((ROWS_PER_S,), jnp.float32),
        pltpu.SemaphoreType.DMA,
        pltpu.SemaphoreType.DMA,
        pltpu.VMEM_SHARED((NP,), jnp.float32),
        pltpu.VMEM_SHARED((NP,), jnp.float32),
    ],
)
def _deg_kernel(ei_hbm, degs_hbm, isrc_v, idst_v, ones_v, zeros_v,
                sem_s, sem_d, dout_sh, din_sh):
    cid, sid, wid = _worker_ids()
    pltpu.async_copy(ei_hbm.at[0, pl.ds(wid * CH_PER_W, CH_PER_W)], isrc_v,
                     sem_s)
    pltpu.async_copy(ei_hbm.at[1, pl.ds(wid * CH_PER_W, CH_PER_W)], idst_v,
                     sem_d)

    def fill_ones(i, _):
        ones_v[pl.ds(i * 16, 16)] = jnp.ones((16,), jnp.float32)
        return 0

    lax.fori_loop(0, CHUNK // 16, fill_ones, 0)

    def fill_zeros(i, _):
        zeros_v[pl.ds(i * 16, 16)] = jnp.zeros((16,), jnp.float32)
        return 0

    lax.fori_loop(0, ROWS_PER_S // 16, fill_zeros, 0)

    base = sid * ROWS_PER_S
    pltpu.sync_copy(zeros_v, dout_sh.at[pl.ds(base, ROWS_PER_S)])
    pltpu.sync_copy(zeros_v, din_sh.at[pl.ds(base, ROWS_PER_S)])
    pltpu.make_async_copy(ei_hbm.at[0, pl.ds(0, CH_PER_W)], isrc_v,
                          sem_s).wait()
    pltpu.make_async_copy(ei_hbm.at[1, pl.ds(0, CH_PER_W)], idst_v,
                          sem_d).wait()
    plsc.subcore_barrier()

    def chunk_body(k, _):
        pltpu.async_copy(ones_v, dout_sh.at[isrc_v.at[k]], sem_s, add=True)
        pltpu.async_copy(ones_v, din_sh.at[idst_v.at[k]], sem_d, add=True)
        pltpu.make_async_copy(ones_v, dout_sh.at[isrc_v.at[k]], sem_s).wait()
        pltpu.make_async_copy(ones_v, din_sh.at[idst_v.at[k]], sem_d).wait()
        return 0

    lax.fori_loop(0, CH_PER_W, chunk_body, 0)
    plsc.subcore_barrier()
    pltpu.sync_copy(dout_sh.at[pl.ds(base, ROWS_PER_S)],
                    degs_hbm.at[0, cid, pl.ds(base, ROWS_PER_S)])
    pltpu.sync_copy(din_sh.at[pl.ds(base, ROWS_PER_S)],
                    degs_hbm.at[1, cid, pl.ds(base, ROWS_PER_S)])


def _make_agg(F):
    @functools.partial(
        pl.kernel,
        out_type=jax.ShapeDtypeStruct((NC, NP, F), jnp.float32),
        mesh=plsc.VectorSubcoreMesh(**_MESH),
        compiler_params=_SC_PARAMS,
        scratch_types=(
            [pltpu.VMEM((CH_PER_W, CHUNK), jnp.int32),
             pltpu.VMEM((CH_PER_W, CHUNK), jnp.int32)]
            + [pltpu.VMEM((CHUNK, F), jnp.float32) for _ in range(NBUF)]
            + [pltpu.SemaphoreType.DMA for _ in range(NBUF)]
            + [pltpu.SemaphoreType.DMA,
               pltpu.VMEM_SHARED((NP, F), jnp.float32)]
        ),
    )
    def _agg_kernel(h_hbm, ei_hbm, out_hbm, isrc_v, idst_v, *rest):
        msgs = rest[:NBUF]
        gsems = rest[NBUF:2 * NBUF]
        isem = rest[2 * NBUF]
        agg_sh = rest[2 * NBUF + 1]
        cid, sid, wid = _worker_ids()
        pltpu.async_copy(ei_hbm.at[0, pl.ds(wid * CH_PER_W, CH_PER_W)],
                         isrc_v, isem)
        pltpu.async_copy(ei_hbm.at[1, pl.ds(wid * CH_PER_W, CH_PER_W)],
                         idst_v, isem)

        def fill_zeros(r, _):
            for l in range(F // 16):
                msgs[0][r, pl.ds(l * 16, 16)] = jnp.zeros((16,), jnp.float32)
            return 0

        lax.fori_loop(0, CHUNK, fill_zeros, 0)
        base = sid * ROWS_PER_S
        for t in range(ROWS_PER_S // CHUNK):
            pltpu.sync_copy(msgs[0], agg_sh.at[pl.ds(base + t * CHUNK, CHUNK)])
        pltpu.make_async_copy(ei_hbm.at[0, pl.ds(0, CH_PER_W)], isrc_v,
                              isem).wait()
        pltpu.make_async_copy(ei_hbm.at[1, pl.ds(0, CH_PER_W)], idst_v,
                              isem).wait()
        plsc.subcore_barrier()

        for j in range(NBUF):
            pltpu.async_copy(h_hbm.at[isrc_v.at[j]], msgs[j], gsems[j])

        def chunk_body(i, _):
            k0 = i * NBUF
            for j in range(NBUF):
                pltpu.make_async_copy(h_hbm.at[isrc_v.at[k0 + j]], msgs[j],
                                      gsems[j]).wait()
                pltpu.sync_copy(msgs[j], agg_sh.at[idst_v.at[k0 + j]],
                                add=True)

                @pl.when(i < CH_PER_W // NBUF - 1)
                def _():
                    pltpu.async_copy(h_hbm.at[isrc_v.at[k0 + NBUF + j]],
                                     msgs[j], gsems[j])

            return 0

        lax.fori_loop(0, CH_PER_W // NBUF, chunk_body, 0)
        plsc.subcore_barrier()
        pltpu.sync_copy(agg_sh.at[pl.ds(base, ROWS_PER_S)],
                        out_hbm.at[cid, pl.ds(base, ROWS_PER_S)])

    return _agg_kernel


_agg_hid = _make_agg(HID)
_agg_out = _make_agg(OUT_P)


def _mm1_body(x_ref, w_ref, o_ref):
    o_ref[...] = jnp.dot(x_ref[...], w_ref[...],
                         preferred_element_type=jnp.float32)


def _scale1_body(u_ref, degs_ref, o_ref):
    d = degs_ref[0, 0, :N] + degs_ref[0, 1, :N]
    s = lax.rsqrt(jnp.maximum(d, 1.0))
    h = u_ref[...] * s[:, None]
    o_ref[...] = jnp.concatenate(
        [h, jnp.zeros((NP - N, HID), jnp.float32)], axis=0)


def _mid_body(a_ref, degs_ref, b1_ref, w2_ref, o_ref):
    agg = a_ref[0, :N, :] + a_ref[1, :N, :]
    din = degs_ref[1, 0, :N] + degs_ref[1, 1, :N]
    dout = degs_ref[0, 0, :N] + degs_ref[0, 1, :N]
    si = lax.rsqrt(jnp.maximum(din, 1.0))
    so = lax.rsqrt(jnp.maximum(dout, 1.0))
    t = jnp.maximum(agg * si[:, None] + b1_ref[0, :][None, :], 0.0)
    h = jnp.dot(t, w2_ref[...],
                preferred_element_type=jnp.float32) * so[:, None]
    o_ref[...] = jnp.concatenate(
        [h, jnp.zeros((NP - N, OUT_P), jnp.float32)], axis=0)


def _out_body(a_ref, degs_ref, b2_ref, o_ref):
    agg = a_ref[0, :N, :OUT_F] + a_ref[1, :N, :OUT_F]
    din = degs_ref[1, 0, :N] + degs_ref[1, 1, :N]
    si = lax.rsqrt(jnp.maximum(din, 1.0))
    o_ref[...] = agg * si[:, None] + b2_ref[0, :][None, :]


_mm1 = pl.pallas_call(
    _mm1_body, out_shape=jax.ShapeDtypeStruct((N, HID), jnp.float32))
_scale1 = pl.pallas_call(
    _scale1_body, out_shape=jax.ShapeDtypeStruct((NP, HID), jnp.float32))
_mid = pl.pallas_call(
    _mid_body, out_shape=jax.ShapeDtypeStruct((NP, OUT_P), jnp.float32))
_out = pl.pallas_call(
    _out_body, out_shape=jax.ShapeDtypeStruct((N, OUT_F), jnp.float32))


def kernel(features, edge_index, W1, b1, W2, b2):
    ei = edge_index.astype(jnp.int32)
    npad = EPAD - E
    sent = (N + (jnp.arange(npad, dtype=jnp.int32) % (NP - N))).reshape(1, npad)
    pad = jnp.concatenate([sent, sent], axis=0)
    ei_pad = jnp.concatenate([ei, pad], axis=1).reshape(2, NCHUNK, CHUNK)

    degs = _deg_kernel(ei_pad)
    u = _mm1(features, W1)
    h1s = _scale1(u, degs)
    agg1 = _agg_hid(h1s, ei_pad)
    w2p = jnp.pad(W2, ((0, 0), (0, OUT_P - OUT_F)))
    h2s = _mid(agg1, degs, b1.reshape(1, -1), w2p)
    agg2 = _agg_out(h2s, ei_pad)
    return _out(agg2, degs, b2.reshape(1, -1))


# speedup vs baseline: 2.8313x; 1.0049x over previous
import functools

import jax
import jax.numpy as jnp
from jax import lax
from jax.experimental import pallas as pl
from jax.experimental.pallas import tpu as pltpu
from jax.experimental.pallas import tpu_sc as plsc

N = 10000
E = 320000
IN_F = 128
HID = 64
OUT_F = 40
OUT_P = 48

NC, NS = 2, 16
NW = NC * NS
ROWS_PER_S = 640
NP = NS * ROWS_PER_S
SENT = N + 100
CHUNK = 128
CH_PER_W = 80
NCHUNK = NW * CH_PER_W
EPAD = NCHUNK * CHUNK
NBUF = 8

_MESH = dict(core_axis_name="c", subcore_axis_name="s")
_SC_PARAMS = pltpu.CompilerParams(use_tc_tiling_on_sc=False)


def _worker_ids():
    cid = lax.axis_index("c")
    sid = lax.axis_index("s")
    return cid, sid, sid * NC + cid


@functools.partial(
    pl.kernel,
    out_type=jax.ShapeDtypeStruct((2, NC, NP), jnp.float32),
    mesh=plsc.VectorSubcoreMesh(**_MESH),
    compiler_params=_SC_PARAMS,
    scratch_types=[
        pltpu.VMEM((CH_PER_W, CHUNK), jnp.int32),
        pltpu.VMEM((CH_PER_W, CHUNK), jnp.int32),
        pltpu.VMEM((CHUNK,), jnp.float32),
        pltpu.VMEM((ROWS_PER_S,), jnp.float32),
        pltpu.SemaphoreType.DMA,
        pltpu.SemaphoreType.DMA,
        pltpu.VMEM_SHARED((NP,), jnp.float32),
        pltpu.VMEM_SHARED((NP,), jnp.float32),
    ],
)
def _deg_kernel(ei_hbm, degs_hbm, isrc_v, idst_v, ones_v, zeros_v,
                sem_s, sem_d, dout_sh, din_sh):
    cid, sid, wid = _worker_ids()
    pltpu.async_copy(ei_hbm.at[0, pl.ds(wid * CH_PER_W, CH_PER_W)], isrc_v,
                     sem_s)
    pltpu.async_copy(ei_hbm.at[1, pl.ds(wid * CH_PER_W, CH_PER_W)], idst_v,
                     sem_d)

    def fill_ones(i, _):
        ones_v[pl.ds(i * 16, 16)] = jnp.ones((16,), jnp.float32)
        return 0

    lax.fori_loop(0, CHUNK // 16, fill_ones, 0)

    def fill_zeros(i, _):
        zeros_v[pl.ds(i * 16, 16)] = jnp.zeros((16,), jnp.float32)
        return 0

    lax.fori_loop(0, ROWS_PER_S // 16, fill_zeros, 0)

    base = sid * ROWS_PER_S
    pltpu.sync_copy(zeros_v, dout_sh.at[pl.ds(base, ROWS_PER_S)])
    pltpu.sync_copy(zeros_v, din_sh.at[pl.ds(base, ROWS_PER_S)])
    pltpu.make_async_copy(ei_hbm.at[0, pl.ds(0, CH_PER_W)], isrc_v,
                          sem_s).wait()
    pltpu.make_async_copy(ei_hbm.at[1, pl.ds(0, CH_PER_W)], idst_v,
                          sem_d).wait()
    plsc.subcore_barrier()

    def chunk_body(k, _):
        pltpu.async_copy(ones_v, dout_sh.at[isrc_v.at[k]], sem_s, add=True)
        pltpu.async_copy(ones_v, din_sh.at[idst_v.at[k]], sem_d, add=True)
        pltpu.make_async_copy(ones_v, dout_sh.at[isrc_v.at[k]], sem_s).wait()
        pltpu.make_async_copy(ones_v, din_sh.at[idst_v.at[k]], sem_d).wait()
        return 0

    lax.fori_loop(0, CH_PER_W, chunk_body, 0)
    plsc.subcore_barrier()
    pltpu.sync_copy(dout_sh.at[pl.ds(base, ROWS_PER_S)],
                    degs_hbm.at[0, cid, pl.ds(base, ROWS_PER_S)])
    pltpu.sync_copy(din_sh.at[pl.ds(base, ROWS_PER_S)],
                    degs_hbm.at[1, cid, pl.ds(base, ROWS_PER_S)])


def _make_agg(F):
    @functools.partial(
        pl.kernel,
        out_type=jax.ShapeDtypeStruct((NC, NP, F), jnp.float32),
        mesh=plsc.VectorSubcoreMesh(**_MESH),
        compiler_params=_SC_PARAMS,
        scratch_types=(
            [pltpu.VMEM((CH_PER_W, CHUNK), jnp.int32),
             pltpu.VMEM((CH_PER_W, CHUNK), jnp.int32)]
            + [pltpu.VMEM((CHUNK, F), jnp.float32) for _ in range(NBUF)]
            + [pltpu.SemaphoreType.DMA for _ in range(NBUF)]
            + [pltpu.SemaphoreType.DMA,
               pltpu.VMEM_SHARED((NP, F), jnp.float32)]
        ),
    )
    def _agg_kernel(h_hbm, ei_hbm, out_hbm, isrc_v, idst_v, *rest):
        msgs = rest[:NBUF]
        gsems = rest[NBUF:2 * NBUF]
        isem = rest[2 * NBUF]
        agg_sh = rest[2 * NBUF + 1]
        cid, sid, wid = _worker_ids()
        pltpu.async_copy(ei_hbm.at[0, pl.ds(wid * CH_PER_W, CH_PER_W)],
                         isrc_v, isem)
        pltpu.async_copy(ei_hbm.at[1, pl.ds(wid * CH_PER_W, CH_PER_W)],
                         idst_v, isem)

        def fill_zeros(r, _):
            for l in range(F // 16):
                msgs[0][r, pl.ds(l * 16, 16)] = jnp.zeros((16,), jnp.float32)
            return 0

        lax.fori_loop(0, CHUNK, fill_zeros, 0)
        base = sid * ROWS_PER_S
        for t in range(ROWS_PER_S // CHUNK):
            pltpu.sync_copy(msgs[0], agg_sh.at[pl.ds(base + t * CHUNK, CHUNK)])
        pltpu.make_async_copy(ei_hbm.at[0, pl.ds(0, CH_PER_W)], isrc_v,
                              isem).wait()
        pltpu.make_async_copy(ei_hbm.at[1, pl.ds(0, CH_PER_W)], idst_v,
                              isem).wait()
        plsc.subcore_barrier()

        for j in range(NBUF):
            pltpu.async_copy(h_hbm.at[isrc_v.at[j]], msgs[j], gsems[j])

        def chunk_body(i, _):
            k0 = i * NBUF
            for j in range(NBUF):
                pltpu.make_async_copy(h_hbm.at[isrc_v.at[k0 + j]], msgs[j],
                                      gsems[j]).wait()
                pltpu.sync_copy(msgs[j], agg_sh.at[idst_v.at[k0 + j]],
                                add=True)

                @pl.when(i < CH_PER_W // NBUF - 1)
                def _():
                    pltpu.async_copy(h_hbm.at[isrc_v.at[k0 + NBUF + j]],
                                     msgs[j], gsems[j])

            return 0

        lax.fori_loop(0, CH_PER_W // NBUF, chunk_body, 0)
        plsc.subcore_barrier()
        pltpu.sync_copy(agg_sh.at[pl.ds(base, ROWS_PER_S)],
                        out_hbm.at[cid, pl.ds(base, ROWS_PER_S)])

    return _agg_kernel


_agg_hid = _make_agg(HID)
_agg_out = _make_agg(OUT_P)


def _mm1_body(x_ref, w_ref, o_ref):
    o_ref[...] = jnp.dot(x_ref[...], w_ref[...],
                         preferred_element_type=jnp.float32)


def _scale1_body(u_ref, degs_ref, o_ref):
    d = degs_ref[0, 0, :N] + degs_ref[0, 1, :N]
    s = lax.rsqrt(jnp.maximum(d, 1.0))
    h = u_ref[...] * s[:, None]
    o_ref[...] = jnp.concatenate(
        [h, jnp.zeros((NP - N, HID), jnp.float32)], axis=0)


def _mid_body(a_ref, degs_ref, b1_ref, w2_ref, o_ref):
    agg = a_ref[0, :N, :] + a_ref[1, :N, :]
    din = degs_ref[1, 0, :N] + degs_ref[1, 1, :N]
    dout = degs_ref[0, 0, :N] + degs_ref[0, 1, :N]
    si = lax.rsqrt(jnp.maximum(din, 1.0))
    so = lax.rsqrt(jnp.maximum(dout, 1.0))
    t = jnp.maximum(agg * si[:, None] + b1_ref[0, :][None, :], 0.0)
    h = jnp.dot(t, w2_ref[...],
                preferred_element_type=jnp.float32) * so[:, None]
    o_ref[...] = jnp.concatenate(
        [h, jnp.zeros((NP - N, OUT_P), jnp.float32)], axis=0)


def _out_body(a_ref, degs_ref, b2_ref, o_ref):
    agg = a_ref[0, :N, :OUT_F] + a_ref[1, :N, :OUT_F]
    din = degs_ref[1, 0, :N] + degs_ref[1, 1, :N]
    si = lax.rsqrt(jnp.maximum(din, 1.0))
    o_ref[...] = agg * si[:, None] + b2_ref[0, :][None, :]


_mm1 = pl.pallas_call(
    _mm1_body, out_shape=jax.ShapeDtypeStruct((N, HID), jnp.float32))
_scale1 = pl.pallas_call(
    _scale1_body, out_shape=jax.ShapeDtypeStruct((NP, HID), jnp.float32))
_mid = pl.pallas_call(
    _mid_body, out_shape=jax.ShapeDtypeStruct((NP, OUT_P), jnp.float32))
_out = pl.pallas_call(
    _out_body, out_shape=jax.ShapeDtypeStruct((N, OUT_F), jnp.float32))


def kernel(features, edge_index, W1, b1, W2, b2):
    ei = edge_index.astype(jnp.int32)
    npad = EPAD - E
    sent = (N + (jnp.arange(npad, dtype=jnp.int32) % (NP - N))).reshape(1, npad)
    pad = jnp.concatenate([sent, sent], axis=0)
    ei_pad = jnp.concatenate([ei, pad], axis=1).reshape(2, NCHUNK, CHUNK)

    degs = _deg_kernel(ei_pad)
    u = _mm1(features, W1)
    h1s = _scale1(u, degs)
    agg1 = _agg_hid(h1s, ei_pad)
    w2p = jnp.pad(W2, ((0, 0), (0, OUT_P - OUT_F)))
    h2s = _mid(agg1, degs, b1.reshape(1, -1), w2p)
    agg2 = _agg_out(h2s, ei_pad)
    return _out(agg2, degs, b2.reshape(1, -1))


# speedup vs baseline: 2.9351x; 1.0367x over previous
import functools

import jax
import jax.numpy as jnp
from jax import lax
from jax.experimental import pallas as pl
from jax.experimental.pallas import tpu as pltpu
from jax.experimental.pallas import tpu_sc as plsc

N = 10000
E = 320000
IN_F = 128
HID = 64
OUT_F = 40
OUT_P = 48

NC, NS = 2, 16
NW = NC * NS
ROWS_PER_S = 640
NP = NS * ROWS_PER_S
SENT = N + 100
CHUNK = 128
CH_PER_W = 80
NCHUNK = NW * CH_PER_W
EPAD = NCHUNK * CHUNK
NBUF = 8

_MESH = dict(core_axis_name="c", subcore_axis_name="s")
_SC_PARAMS = pltpu.CompilerParams(use_tc_tiling_on_sc=False)


def _worker_ids():
    cid = lax.axis_index("c")
    sid = lax.axis_index("s")
    return cid, sid, sid * NC + cid


NCHUNK_RAW = E // CHUNK
CH_LO = NCHUNK_RAW // NW
CH_EXTRA = NCHUNK_RAW - CH_LO * NW


@functools.partial(
    pl.kernel,
    out_type=jax.ShapeDtypeStruct((2, NC, NP), jnp.float32),
    mesh=plsc.VectorSubcoreMesh(**_MESH),
    compiler_params=pltpu.CompilerParams(use_tc_tiling_on_sc=True),
    scratch_types=[
        pltpu.VMEM((CH_LO + 1, CHUNK), jnp.int32),
        pltpu.VMEM((CH_LO + 1, CHUNK), jnp.int32),
        pltpu.VMEM((CHUNK,), jnp.float32),
        pltpu.VMEM((ROWS_PER_S,), jnp.float32),
        pltpu.SemaphoreType.DMA,
        pltpu.SemaphoreType.DMA,
        pltpu.VMEM_SHARED((NP,), jnp.float32),
        pltpu.VMEM_SHARED((NP,), jnp.float32),
    ],
)
def _deg_kernel(ei_hbm, degs_hbm, isrc_v, idst_v, ones_v, zeros_v,
                sem_s, sem_d, dout_sh, din_sh):
    cid, sid, wid = _worker_ids()
    start = wid * CH_LO + jnp.minimum(wid, CH_EXTRA)
    nch = CH_LO + jnp.where(wid < CH_EXTRA, 1, 0)

    def fetch_body(k, _):
        @pl.when(k < nch)
        def _():
            pltpu.async_copy(ei_hbm.at[0, pl.ds((start + k) * CHUNK, CHUNK)],
                             isrc_v.at[k], sem_s)
            pltpu.async_copy(ei_hbm.at[1, pl.ds((start + k) * CHUNK, CHUNK)],
                             idst_v.at[k], sem_d)
        return 0

    lax.fori_loop(0, CH_LO + 1, fetch_body, 0)

    def fill_ones(i, _):
        ones_v[pl.ds(i * 16, 16)] = jnp.ones((16,), jnp.float32)
        return 0

    lax.fori_loop(0, CHUNK // 16, fill_ones, 0)

    def fill_zeros(i, _):
        zeros_v[pl.ds(i * 16, 16)] = jnp.zeros((16,), jnp.float32)
        return 0

    lax.fori_loop(0, ROWS_PER_S // 16, fill_zeros, 0)

    base = sid * ROWS_PER_S
    pltpu.sync_copy(zeros_v, dout_sh.at[pl.ds(base, ROWS_PER_S)])
    pltpu.sync_copy(zeros_v, din_sh.at[pl.ds(base, ROWS_PER_S)])

    def drain_body(k, _):
        @pl.when(k < nch)
        def _():
            pltpu.make_async_copy(ei_hbm.at[0, pl.ds(0, CHUNK)],
                                  isrc_v.at[k], sem_s).wait()
            pltpu.make_async_copy(ei_hbm.at[1, pl.ds(0, CHUNK)],
                                  idst_v.at[k], sem_d).wait()
        return 0

    lax.fori_loop(0, CH_LO + 1, drain_body, 0)
    plsc.subcore_barrier()

    def chunk_body(k, _):
        @pl.when(k < nch)
        def _():
            pltpu.async_copy(ones_v, dout_sh.at[isrc_v.at[k]], sem_s,
                             add=True)
            pltpu.async_copy(ones_v, din_sh.at[idst_v.at[k]], sem_d,
                             add=True)
            pltpu.make_async_copy(ones_v, dout_sh.at[isrc_v.at[k]],
                                  sem_s).wait()
            pltpu.make_async_copy(ones_v, din_sh.at[idst_v.at[k]],
                                  sem_d).wait()
        return 0

    lax.fori_loop(0, CH_LO + 1, chunk_body, 0)
    plsc.subcore_barrier()
    pltpu.sync_copy(dout_sh.at[pl.ds(base, ROWS_PER_S)],
                    degs_hbm.at[0, cid, pl.ds(base, ROWS_PER_S)])
    pltpu.sync_copy(din_sh.at[pl.ds(base, ROWS_PER_S)],
                    degs_hbm.at[1, cid, pl.ds(base, ROWS_PER_S)])


def _make_agg(F):
    @functools.partial(
        pl.kernel,
        out_type=jax.ShapeDtypeStruct((NC, NP, F), jnp.float32),
        mesh=plsc.VectorSubcoreMesh(**_MESH),
        compiler_params=_SC_PARAMS,
        scratch_types=(
            [pltpu.VMEM((CH_PER_W, CHUNK), jnp.int32),
             pltpu.VMEM((CH_PER_W, CHUNK), jnp.int32)]
            + [pltpu.VMEM((CHUNK, F), jnp.float32) for _ in range(NBUF)]
            + [pltpu.SemaphoreType.DMA for _ in range(NBUF)]
            + [pltpu.SemaphoreType.DMA,
               pltpu.VMEM_SHARED((NP, F), jnp.float32)]
        ),
    )
    def _agg_kernel(h_hbm, ei_hbm, out_hbm, isrc_v, idst_v, *rest):
        msgs = rest[:NBUF]
        gsems = rest[NBUF:2 * NBUF]
        isem = rest[2 * NBUF]
        agg_sh = rest[2 * NBUF + 1]
        cid, sid, wid = _worker_ids()
        pltpu.async_copy(ei_hbm.at[0, pl.ds(wid * CH_PER_W, CH_PER_W)],
                         isrc_v, isem)
        pltpu.async_copy(ei_hbm.at[1, pl.ds(wid * CH_PER_W, CH_PER_W)],
                         idst_v, isem)

        def fill_zeros(r, _):
            for l in range(F // 16):
                msgs[0][r, pl.ds(l * 16, 16)] = jnp.zeros((16,), jnp.float32)
            return 0

        lax.fori_loop(0, CHUNK, fill_zeros, 0)
        base = sid * ROWS_PER_S
        for t in range(ROWS_PER_S // CHUNK):
            pltpu.sync_copy(msgs[0], agg_sh.at[pl.ds(base + t * CHUNK, CHUNK)])
        pltpu.make_async_copy(ei_hbm.at[0, pl.ds(0, CH_PER_W)], isrc_v,
                              isem).wait()
        pltpu.make_async_copy(ei_hbm.at[1, pl.ds(0, CH_PER_W)], idst_v,
                              isem).wait()
        plsc.subcore_barrier()

        for j in range(NBUF):
            pltpu.async_copy(h_hbm.at[isrc_v.at[j]], msgs[j], gsems[j])

        def chunk_body(i, _):
            k0 = i * NBUF
            for j in range(NBUF):
                pltpu.make_async_copy(h_hbm.at[isrc_v.at[k0 + j]], msgs[j],
                                      gsems[j]).wait()
                pltpu.sync_copy(msgs[j], agg_sh.at[idst_v.at[k0 + j]],
                                add=True)

                @pl.when(i < CH_PER_W // NBUF - 1)
                def _():
                    pltpu.async_copy(h_hbm.at[isrc_v.at[k0 + NBUF + j]],
                                     msgs[j], gsems[j])

            return 0

        lax.fori_loop(0, CH_PER_W // NBUF, chunk_body, 0)
        plsc.subcore_barrier()
        pltpu.sync_copy(agg_sh.at[pl.ds(base, ROWS_PER_S)],
                        out_hbm.at[cid, pl.ds(base, ROWS_PER_S)])

    return _agg_kernel


_agg_hid = _make_agg(HID)
_agg_out = _make_agg(OUT_P)


def _mm1_body(x_ref, w_ref, o_ref):
    o_ref[...] = jnp.dot(x_ref[...], w_ref[...],
                         preferred_element_type=jnp.float32)


def _scale1_body(u_ref, degs_ref, o_ref):
    d = degs_ref[0, 0, :N] + degs_ref[0, 1, :N]
    s = lax.rsqrt(jnp.maximum(d, 1.0))
    h = u_ref[...] * s[:, None]
    o_ref[...] = jnp.concatenate(
        [h, jnp.zeros((NP - N, HID), jnp.float32)], axis=0)


def _mid_body(a_ref, degs_ref, b1_ref, w2_ref, o_ref):
    agg = a_ref[0, :N, :] + a_ref[1, :N, :]
    din = degs_ref[1, 0, :N] + degs_ref[1, 1, :N]
    dout = degs_ref[0, 0, :N] + degs_ref[0, 1, :N]
    si = lax.rsqrt(jnp.maximum(din, 1.0))
    so = lax.rsqrt(jnp.maximum(dout, 1.0))
    t = jnp.maximum(agg * si[:, None] + b1_ref[0, :][None, :], 0.0)
    h = jnp.dot(t, w2_ref[...],
                preferred_element_type=jnp.float32) * so[:, None]
    o_ref[...] = jnp.concatenate(
        [h, jnp.zeros((NP - N, OUT_P), jnp.float32)], axis=0)


def _out_body(a_ref, degs_ref, b2_ref, o_ref):
    agg = a_ref[0, :N, :OUT_F] + a_ref[1, :N, :OUT_F]
    din = degs_ref[1, 0, :N] + degs_ref[1, 1, :N]
    si = lax.rsqrt(jnp.maximum(din, 1.0))
    o_ref[...] = agg * si[:, None] + b2_ref[0, :][None, :]


_mm1 = pl.pallas_call(
    _mm1_body, out_shape=jax.ShapeDtypeStruct((N, HID), jnp.float32))
_scale1 = pl.pallas_call(
    _scale1_body, out_shape=jax.ShapeDtypeStruct((NP, HID), jnp.float32))
_mid = pl.pallas_call(
    _mid_body, out_shape=jax.ShapeDtypeStruct((NP, OUT_P), jnp.float32))
_out = pl.pallas_call(
    _out_body, out_shape=jax.ShapeDtypeStruct((N, OUT_F), jnp.float32))


def kernel(features, edge_index, W1, b1, W2, b2):
    ei = edge_index.astype(jnp.int32)
    npad = EPAD - E
    sent = (N + (jnp.arange(npad, dtype=jnp.int32) % (NP - N))).reshape(1, npad)
    pad = jnp.concatenate([sent, sent], axis=0)
    ei_pad = jnp.concatenate([ei, pad], axis=1).reshape(2, NCHUNK, CHUNK)

    degs = _deg_kernel(ei)
    u = _mm1(features, W1)
    h1s = _scale1(u, degs)
    agg1 = _agg_hid(h1s, ei_pad)
    w2p = jnp.pad(W2, ((0, 0), (0, OUT_P - OUT_F)))
    h2s = _mid(agg1, degs, b1.reshape(1, -1), w2p)
    agg2 = _agg_out(h2s, ei_pad)
    return _out(agg2, degs, b2.reshape(1, -1))


# speedup vs baseline: 2.9847x; 1.0169x over previous
import functools

import jax
import jax.numpy as jnp
from jax import lax
from jax.experimental import pallas as pl
from jax.experimental.pallas import tpu as pltpu
from jax.experimental.pallas import tpu_sc as plsc

N = 10000
E = 320000
IN_F = 128
HID = 64
OUT_F = 40
OUT_P = 48

NC, NS = 2, 16
NW = NC * NS
ROWS_PER_S = 640
NP = NS * ROWS_PER_S
CHUNK = 128
CH_PER_W = 80
NCHUNK = NW * CH_PER_W
EPAD = NCHUNK * CHUNK
NBUF = 8

_MESH = dict(core_axis_name="c", subcore_axis_name="s")
_SC_PARAMS = pltpu.CompilerParams(use_tc_tiling_on_sc=False)


def _worker_ids():
    cid = lax.axis_index("c")
    sid = lax.axis_index("s")
    return cid, sid, sid * NC + cid


NCHUNK_RAW = E // CHUNK
CH_LO = NCHUNK_RAW // NW
CH_EXTRA = NCHUNK_RAW - CH_LO * NW


@functools.partial(
    pl.kernel,
    out_type=jax.ShapeDtypeStruct((2, NC, NP), jnp.float32),
    mesh=plsc.VectorSubcoreMesh(**_MESH),
    compiler_params=pltpu.CompilerParams(use_tc_tiling_on_sc=True),
    scratch_types=[
        pltpu.VMEM((CH_LO + 1, CHUNK), jnp.int32),
        pltpu.VMEM((CH_LO + 1, CHUNK), jnp.int32),
        pltpu.VMEM((CHUNK,), jnp.float32),
        pltpu.VMEM((ROWS_PER_S,), jnp.float32),
        pltpu.SemaphoreType.DMA,
        pltpu.SemaphoreType.DMA,
        pltpu.VMEM_SHARED((NP,), jnp.float32),
        pltpu.VMEM_SHARED((NP,), jnp.float32),
    ],
)
def _deg_kernel(ei_hbm, degs_hbm, isrc_v, idst_v, ones_v, zeros_v,
                sem_s, sem_d, dout_sh, din_sh):
    cid, sid, wid = _worker_ids()
    start = wid * CH_LO + jnp.minimum(wid, CH_EXTRA)
    nch = CH_LO + jnp.where(wid < CH_EXTRA, 1, 0)

    def fetch_body(k, _):
        @pl.when(k < nch)
        def _():
            pltpu.async_copy(ei_hbm.at[0, pl.ds((start + k) * CHUNK, CHUNK)],
                             isrc_v.at[k], sem_s)
            pltpu.async_copy(ei_hbm.at[1, pl.ds((start + k) * CHUNK, CHUNK)],
                             idst_v.at[k], sem_d)
        return 0

    lax.fori_loop(0, CH_LO + 1, fetch_body, 0)

    def fill_ones(i, _):
        ones_v[pl.ds(i * 16, 16)] = jnp.ones((16,), jnp.float32)
        return 0

    lax.fori_loop(0, CHUNK // 16, fill_ones, 0)

    def fill_zeros(i, _):
        zeros_v[pl.ds(i * 16, 16)] = jnp.zeros((16,), jnp.float32)
        return 0

    lax.fori_loop(0, ROWS_PER_S // 16, fill_zeros, 0)

    base = sid * ROWS_PER_S
    pltpu.sync_copy(zeros_v, dout_sh.at[pl.ds(base, ROWS_PER_S)])
    pltpu.sync_copy(zeros_v, din_sh.at[pl.ds(base, ROWS_PER_S)])

    def drain_body(k, _):
        @pl.when(k < nch)
        def _():
            pltpu.make_async_copy(ei_hbm.at[0, pl.ds(0, CHUNK)],
                                  isrc_v.at[k], sem_s).wait()
            pltpu.make_async_copy(ei_hbm.at[1, pl.ds(0, CHUNK)],
                                  idst_v.at[k], sem_d).wait()
        return 0

    lax.fori_loop(0, CH_LO + 1, drain_body, 0)
    plsc.subcore_barrier()

    pltpu.async_copy(ones_v, dout_sh.at[isrc_v.at[0]], sem_s, add=True)
    pltpu.async_copy(ones_v, din_sh.at[idst_v.at[0]], sem_d, add=True)

    def chunk_body(k, _):
        @pl.when(k < nch)
        def _():
            pltpu.async_copy(ones_v, dout_sh.at[isrc_v.at[k]], sem_s,
                             add=True)
            pltpu.async_copy(ones_v, din_sh.at[idst_v.at[k]], sem_d,
                             add=True)

        @pl.when(k <= nch)
        def _():
            pltpu.make_async_copy(ones_v, dout_sh.at[isrc_v.at[0]],
                                  sem_s).wait()
            pltpu.make_async_copy(ones_v, din_sh.at[idst_v.at[0]],
                                  sem_d).wait()
        return 0

    lax.fori_loop(1, CH_LO + 1, chunk_body, 0)

    @pl.when(CH_LO < nch)
    def _():
        pltpu.make_async_copy(ones_v, dout_sh.at[isrc_v.at[0]],
                              sem_s).wait()
        pltpu.make_async_copy(ones_v, din_sh.at[idst_v.at[0]],
                              sem_d).wait()

    plsc.subcore_barrier()
    pltpu.sync_copy(dout_sh.at[pl.ds(base, ROWS_PER_S)],
                    degs_hbm.at[0, cid, pl.ds(base, ROWS_PER_S)])
    pltpu.sync_copy(din_sh.at[pl.ds(base, ROWS_PER_S)],
                    degs_hbm.at[1, cid, pl.ds(base, ROWS_PER_S)])


def _make_agg(F):
    @functools.partial(
        pl.kernel,
        out_type=jax.ShapeDtypeStruct((NC, NP, F), jnp.float32),
        mesh=plsc.VectorSubcoreMesh(**_MESH),
        compiler_params=_SC_PARAMS,
        scratch_types=(
            [pltpu.VMEM((CH_PER_W, CHUNK), jnp.int32),
             pltpu.VMEM((CH_PER_W, CHUNK), jnp.int32)]
            + [pltpu.VMEM((CHUNK, F), jnp.float32) for _ in range(NBUF)]
            + [pltpu.SemaphoreType.DMA for _ in range(NBUF)]
            + [pltpu.SemaphoreType.DMA,
               pltpu.VMEM_SHARED((NP, F), jnp.float32)]
        ),
    )
    def _agg_kernel(h_hbm, ei_hbm, out_hbm, isrc_v, idst_v, *rest):
        msgs = rest[:NBUF]
        gsems = rest[NBUF:2 * NBUF]
        isem = rest[2 * NBUF]
        agg_sh = rest[2 * NBUF + 1]
        cid, sid, wid = _worker_ids()
        pltpu.async_copy(ei_hbm.at[0, pl.ds(wid * CH_PER_W, CH_PER_W)],
                         isrc_v, isem)
        pltpu.async_copy(ei_hbm.at[1, pl.ds(wid * CH_PER_W, CH_PER_W)],
                         idst_v, isem)

        def fill_zeros(r, _):
            for l in range(F // 16):
                msgs[0][r, pl.ds(l * 16, 16)] = jnp.zeros((16,), jnp.float32)
            return 0

        lax.fori_loop(0, CHUNK, fill_zeros, 0)
        base = sid * ROWS_PER_S
        for t in range(ROWS_PER_S // CHUNK):
            pltpu.sync_copy(msgs[0], agg_sh.at[pl.ds(base + t * CHUNK, CHUNK)])
        pltpu.make_async_copy(ei_hbm.at[0, pl.ds(0, CH_PER_W)], isrc_v,
                              isem).wait()
        pltpu.make_async_copy(ei_hbm.at[1, pl.ds(0, CH_PER_W)], idst_v,
                              isem).wait()
        plsc.subcore_barrier()

        for j in range(NBUF):
            pltpu.async_copy(h_hbm.at[isrc_v.at[j]], msgs[j], gsems[j])

        def chunk_body(i, _):
            k0 = i * NBUF
            for j in range(NBUF):
                pltpu.make_async_copy(h_hbm.at[isrc_v.at[k0 + j]], msgs[j],
                                      gsems[j]).wait()
                pltpu.sync_copy(msgs[j], agg_sh.at[idst_v.at[k0 + j]],
                                add=True)

                @pl.when(i < CH_PER_W // NBUF - 1)
                def _():
                    pltpu.async_copy(h_hbm.at[isrc_v.at[k0 + NBUF + j]],
                                     msgs[j], gsems[j])

            return 0

        lax.fori_loop(0, CH_PER_W // NBUF, chunk_body, 0)
        plsc.subcore_barrier()
        pltpu.sync_copy(agg_sh.at[pl.ds(base, ROWS_PER_S)],
                        out_hbm.at[cid, pl.ds(base, ROWS_PER_S)])

    return _agg_kernel


_agg_hid = _make_agg(HID)
_agg_out = _make_agg(OUT_P)


def _mm1_body(x_ref, w_ref, o_ref):
    o_ref[...] = jnp.dot(x_ref[...], w_ref[...],
                         preferred_element_type=jnp.float32)


def _scale1_body(u_ref, degs_ref, o_ref):
    d = degs_ref[0, 0, :N] + degs_ref[0, 1, :N]
    s = lax.rsqrt(jnp.maximum(d, 1.0))
    h = u_ref[...] * s[:, None]
    o_ref[...] = jnp.concatenate(
        [h, jnp.zeros((NP - N, HID), jnp.float32)], axis=0)


def _mid_body(a_ref, degs_ref, b1_ref, w2_ref, o_ref):
    agg = a_ref[0, :N, :] + a_ref[1, :N, :]
    din = degs_ref[1, 0, :N] + degs_ref[1, 1, :N]
    dout = degs_ref[0, 0, :N] + degs_ref[0, 1, :N]
    si = lax.rsqrt(jnp.maximum(din, 1.0))
    so = lax.rsqrt(jnp.maximum(dout, 1.0))
    t = jnp.maximum(agg * si[:, None] + b1_ref[0, :][None, :], 0.0)
    h = jnp.dot(t, w2_ref[...],
                preferred_element_type=jnp.float32) * so[:, None]
    o_ref[...] = jnp.concatenate(
        [h, jnp.zeros((NP - N, OUT_P), jnp.float32)], axis=0)


def _out_body(a_ref, degs_ref, b2_ref, o_ref):
    agg = a_ref[0, :N, :OUT_F] + a_ref[1, :N, :OUT_F]
    din = degs_ref[1, 0, :N] + degs_ref[1, 1, :N]
    si = lax.rsqrt(jnp.maximum(din, 1.0))
    o_ref[...] = agg * si[:, None] + b2_ref[0, :][None, :]


_mm1 = pl.pallas_call(
    _mm1_body, out_shape=jax.ShapeDtypeStruct((N, HID), jnp.float32))
_scale1 = pl.pallas_call(
    _scale1_body, out_shape=jax.ShapeDtypeStruct((NP, HID), jnp.float32))
_mid = pl.pallas_call(
    _mid_body, out_shape=jax.ShapeDtypeStruct((NP, OUT_P), jnp.float32))
_out = pl.pallas_call(
    _out_body, out_shape=jax.ShapeDtypeStruct((N, OUT_F), jnp.float32))


def kernel(features, edge_index, W1, b1, W2, b2):
    ei = edge_index.astype(jnp.int32)
    npad = EPAD - E
    sent = (N + (jnp.arange(npad, dtype=jnp.int32) % (NP - N))).reshape(1, npad)
    pad = jnp.concatenate([sent, sent], axis=0)
    ei_pad = jnp.concatenate([ei, pad], axis=1).reshape(2, NCHUNK, CHUNK)

    degs = _deg_kernel(ei)
    u = _mm1(features, W1)
    h1s = _scale1(u, degs)
    agg1 = _agg_hid(h1s, ei_pad)
    w2p = jnp.pad(W2, ((0, 0), (0, OUT_P - OUT_F)))
    h2s = _mid(agg1, degs, b1.reshape(1, -1), w2p)
    agg2 = _agg_out(h2s, ei_pad)
    return _out(agg2, degs, b2.reshape(1, -1))
